# baseline (speedup 1.0000x reference)
import functools

import jax
import jax.numpy as jnp
from jax import lax
from jax.experimental import pallas as pl
from jax.experimental.pallas import tpu as pltpu
from jax.experimental.pallas import tpu_sc as plsc

B = 16384
NF = 26
ED = 10
EDP = 16
FEAT = NF * ED
FEATP = NF * EDP
NC, NS = 2, 16
NW = NC * NS
DEEP = B * NF
PER_W = DEEP // NW
G = 128
KTOT = PER_W // G
NCH = 8
KCH = KTOT // NCH
CH = PER_W // NCH
WPER = B // NW
KW = WPER // G
LRR = 1000000 // EDP


def _sc_gather(idx_d, idx_w, stable16, lr16):
    mesh = plsc.VectorSubcoreMesh(
        core_axis_name="c", subcore_axis_name="s", num_cores=NC, num_subcores=NS
    )

    @functools.partial(
        pl.kernel,
        out_type=(
            jax.ShapeDtypeStruct((DEEP, EDP), jnp.float32),
            jax.ShapeDtypeStruct((B, EDP), jnp.float32),
        ),
        mesh=mesh,
        compiler_params=pltpu.CompilerParams(use_tc_tiling_on_sc=False),
        scratch_types=(
            pltpu.VMEM((KTOT, G), jnp.int32),
            pltpu.VMEM((CH, EDP), jnp.float32),
            pltpu.VMEM((KW, G), jnp.int32),
            pltpu.VMEM((WPER, EDP), jnp.float32),
            pltpu.SemaphoreType.DMA,
        ),
    )
    def run(idx_d_hbm, idx_w_hbm, st_hbm, lr_hbm, out_e, out_w, idx_v, ebuf, widx_v, wbuf, sem):
        wid = lax.axis_index("s") * NC + lax.axis_index("c")
        pltpu.sync_copy(idx_d_hbm.at[wid], idx_v)
        pltpu.sync_copy(idx_w_hbm.at[wid], widx_v)

        wcps = [
            pltpu.make_async_copy(lr_hbm.at[widx_v.at[j]], wbuf.at[pl.ds(j * G, G)], sem)
            for j in range(KW)
        ]
        for cp in wcps:
            cp.start()
        for cp in wcps:
            cp.wait()
        pltpu.sync_copy(wbuf, out_w.at[pl.ds(wid * WPER, WPER)])

        base = wid * PER_W

        def chunk(c, carry):
            cps = [
                pltpu.make_async_copy(
                    st_hbm.at[idx_v.at[c * KCH + j]], ebuf.at[pl.ds(j * G, G)], sem
                )
                for j in range(KCH)
            ]
            for cp in cps:
                cp.start()
            for cp in cps:
                cp.wait()
            pltpu.sync_copy(ebuf, out_e.at[pl.ds(base + c * CH, CH)])
            return carry

        lax.fori_loop(0, NCH, chunk, 0)

    return run(idx_d, idx_w, stable16, lr16)


TB = 1024
NT = B // TB


def _mlp_body(e_ref, w16_ref, lane_ref, w1_ref, b1_ref, w2_ref, b2_ref, w3_ref,
              b3_ref, w4_ref, b4_ref, out_ref, stat_ref):
    p = pl.program_id(0)
    t = pl.program_id(1)

    @pl.when(p == 0)
    def _stats():
        e = e_ref[...]
        s = jnp.sum(e, axis=0, keepdims=True)
        s2 = jnp.sum(e * e, axis=0, keepdims=True)

        @pl.when(t == 0)
        def _init():
            stat_ref[0:1, :] = s
            stat_ref[1:2, :] = s2

        @pl.when(t > 0)
        def _acc():
            stat_ref[0:1, :] += s
            stat_ref[1:2, :] += s2

        @pl.when(t == NT - 1)
        def _fin():
            mean = stat_ref[0:1, :] * (1.0 / B)
            var = stat_ref[1:2, :] * (1.0 / B) - mean * mean
            stat_ref[0:1, :] = mean
            stat_ref[1:2, :] = lax.rsqrt(var + 1e-5)

    @pl.when(p == 1)
    def _mlp():
        mean = stat_ref[0:1, :]
        istd = stat_ref[1:2, :]
        h = ((e_ref[...] - mean) * istd).astype(jnp.bfloat16)
        h = jnp.maximum(jnp.dot(h, w1_ref[...], preferred_element_type=jnp.float32) + b1_ref[...], 0.0).astype(jnp.bfloat16)
        h = jnp.maximum(jnp.dot(h, w2_ref[...], preferred_element_type=jnp.float32) + b2_ref[...], 0.0).astype(jnp.bfloat16)
        h = jnp.maximum(jnp.dot(h, w3_ref[...], preferred_element_type=jnp.float32) + b3_ref[...], 0.0).astype(jnp.bfloat16)
        lane_hit = jax.lax.broadcasted_iota(jnp.int32, (TB, EDP), 1) == lane_ref[...]
        x1 = jnp.sum(jnp.where(lane_hit, w16_ref[...], 0.0), axis=1, keepdims=True)
        z = jnp.dot(h, w4_ref[...], preferred_element_type=jnp.float32) + b4_ref[...] + x1
        out_ref[...] = jax.nn.sigmoid(z)


def _mlp(E, w16, lane, W1, b1, W2, b2, W3, b3, W4, b4):
    return pl.pallas_call(
        _mlp_body,
        grid=(2, NT),
        in_specs=[
            pl.BlockSpec((TB, FEATP), lambda p, t: (t, 0)),
            pl.BlockSpec((TB, EDP), lambda p, t: (t, 0)),
            pl.BlockSpec((TB, 1), lambda p, t: (t, 0)),
            pl.BlockSpec((FEATP, 1024), lambda p, t: (0, 0)),
            pl.BlockSpec((1, 1024), lambda p, t: (0, 0)),
            pl.BlockSpec((1024, 512), lambda p, t: (0, 0)),
            pl.BlockSpec((1, 512), lambda p, t: (0, 0)),
            pl.BlockSpec((512, 256), lambda p, t: (0, 0)),
            pl.BlockSpec((1, 256), lambda p, t: (0, 0)),
            pl.BlockSpec((256, 1), lambda p, t: (0, 0)),
            pl.BlockSpec((1, 1), lambda p, t: (0, 0)),
        ],
        out_specs=pl.BlockSpec((TB, 1), lambda p, t: (t, 0)),
        out_shape=jax.ShapeDtypeStruct((B, 1), jnp.float32),
        scratch_shapes=[pltpu.VMEM((2, FEATP), jnp.float32)],
    )(E, w16, lane, W1, b1, W2, b2, W3, b3, W4, b4)


def kernel(x, lr_table, sparse_table, W1, b1, W2, b2, W3, b3, W4, b4):
    xi = x.astype(jnp.int32)
    idx_d = xi.reshape(NW, KTOT, G)
    x0 = xi[:, 0]
    idx_w = (x0 // EDP).reshape(NW, KW, G)
    lane = (x0 % EDP).reshape(B, 1)
    st16 = jnp.pad(sparse_table, ((0, 0), (0, EDP - ED)))
    lr16 = lr_table[:, 0].reshape(LRR, EDP)
    e_flat, w16 = _sc_gather(idx_d, idx_w, st16, lr16)
    E = e_flat.reshape(B, FEATP)
    W1p = jnp.zeros((FEATP, 1024), W1.dtype).at[
        (jnp.arange(FEAT) // ED) * EDP + (jnp.arange(FEAT) % ED)
    ].set(W1)
    return _mlp(
        E, w16, lane,
        W1p.astype(jnp.bfloat16), b1.reshape(1, 1024),
        W2.astype(jnp.bfloat16), b2.reshape(1, 512),
        W3.astype(jnp.bfloat16), b3.reshape(1, 256),
        W4.astype(jnp.bfloat16), b4.reshape(1, 1),
    )


# speedup vs baseline: 1.1352x; 1.1352x over previous
import functools

import jax
import jax.numpy as jnp
from jax import lax
from jax.experimental import pallas as pl
from jax.experimental.pallas import tpu as pltpu
from jax.experimental.pallas import tpu_sc as plsc

B = 16384
NF = 26
ED = 10
EDP = 16
FEAT = NF * ED
FEATP = NF * EDP
NC, NS = 2, 16
NW = NC * NS
DEEP = B * NF
PER_W = DEEP // NW
G = 128
KTOT = PER_W // G
NCH = 8
KCH = KTOT // NCH
CH = PER_W // NCH
WPER = B // NW
KW = WPER // G
LRR = 1000000 // EDP


def _sc_gather(idx_d, idx_w, stable16, lr16):
    mesh = plsc.VectorSubcoreMesh(
        core_axis_name="c", subcore_axis_name="s", num_cores=NC, num_subcores=NS
    )

    @functools.partial(
        pl.kernel,
        out_type=(
            jax.ShapeDtypeStruct((DEEP, EDP), jnp.float32),
            jax.ShapeDtypeStruct((B, EDP), jnp.float32),
        ),
        mesh=mesh,
        compiler_params=pltpu.CompilerParams(use_tc_tiling_on_sc=False),
        scratch_types=(
            pltpu.VMEM((KTOT, G), jnp.int32),
            pltpu.VMEM((CH, EDP), jnp.float32),
            pltpu.VMEM((KW, G), jnp.int32),
            pltpu.VMEM((WPER, EDP), jnp.float32),
            pltpu.SemaphoreType.DMA,
        ),
    )
    def run(idx_d_hbm, idx_w_hbm, st_hbm, lr_hbm, out_e, out_w, idx_v, ebuf, widx_v, wbuf, sem):
        wid = lax.axis_index("s") * NC + lax.axis_index("c")
        pltpu.sync_copy(idx_d_hbm.at[wid], idx_v)
        pltpu.sync_copy(idx_w_hbm.at[wid], widx_v)

        wcps = [
            pltpu.make_async_copy(lr_hbm.at[widx_v.at[j]], wbuf.at[pl.ds(j * G, G)], sem)
            for j in range(KW)
        ]
        for cp in wcps:
            cp.start()
        for cp in wcps:
            cp.wait()
        pltpu.sync_copy(wbuf, out_w.at[pl.ds(wid * WPER, WPER)])

        base = wid * PER_W

        def chunk(c, carry):
            cps = [
                pltpu.make_async_copy(
                    st_hbm.at[idx_v.at[c * KCH + j]], ebuf.at[pl.ds(j * G, G)], sem
                )
                for j in range(KCH)
            ]
            for cp in cps:
                cp.start()
            for cp in cps:
                cp.wait()
            pltpu.sync_copy(ebuf, out_e.at[pl.ds(base + c * CH, CH)])
            return carry

        lax.fori_loop(0, NCH, chunk, 0)

    return run(idx_d, idx_w, stable16, lr16)


TBV = 4096


def _tp_body(in_ref, out_ref):
    xt = jnp.transpose(in_ref[...], (1, 0))
    out_ref[:, 0:ED] = xt
    out_ref[:, ED:EDP] = jnp.zeros((TBV, EDP - ED), jnp.float32)


def _transpose_pad(st_t):
    nv = st_t.shape[1]
    grid = (nv + TBV - 1) // TBV
    return pl.pallas_call(
        _tp_body,
        grid=(grid,),
        in_specs=[pl.BlockSpec((ED, TBV), lambda i: (0, i))],
        out_specs=pl.BlockSpec((TBV, EDP), lambda i: (i, 0)),
        out_shape=jax.ShapeDtypeStruct((nv, EDP), jnp.float32),
    )(st_t)


TB = 1024
NT = B // TB


def _mlp_body(e_ref, w16_ref, lane_ref, w1_ref, b1_ref, w2_ref, b2_ref, w3_ref,
              b3_ref, w4_ref, b4_ref, out_ref, stat_ref):
    p = pl.program_id(0)
    t = pl.program_id(1)

    @pl.when(p == 0)
    def _stats():
        e = e_ref[...]
        s = jnp.sum(e, axis=0, keepdims=True)
        s2 = jnp.sum(e * e, axis=0, keepdims=True)

        @pl.when(t == 0)
        def _init():
            stat_ref[0:1, :] = s
            stat_ref[1:2, :] = s2

        @pl.when(t > 0)
        def _acc():
            stat_ref[0:1, :] += s
            stat_ref[1:2, :] += s2

        @pl.when(t == NT - 1)
        def _fin():
            mean = stat_ref[0:1, :] * (1.0 / B)
            var = stat_ref[1:2, :] * (1.0 / B) - mean * mean
            stat_ref[0:1, :] = mean
            stat_ref[1:2, :] = lax.rsqrt(var + 1e-5)

    @pl.when(p == 1)
    def _mlp():
        mean = stat_ref[0:1, :]
        istd = stat_ref[1:2, :]
        h = ((e_ref[...] - mean) * istd).astype(jnp.bfloat16)
        h = jnp.maximum(jnp.dot(h, w1_ref[...], preferred_element_type=jnp.float32) + b1_ref[...], 0.0).astype(jnp.bfloat16)
        h = jnp.maximum(jnp.dot(h, w2_ref[...], preferred_element_type=jnp.float32) + b2_ref[...], 0.0).astype(jnp.bfloat16)
        h = jnp.maximum(jnp.dot(h, w3_ref[...], preferred_element_type=jnp.float32) + b3_ref[...], 0.0).astype(jnp.bfloat16)
        lane_hit = jax.lax.broadcasted_iota(jnp.int32, (TB, EDP), 1) == lane_ref[...]
        x1 = jnp.sum(jnp.where(lane_hit, w16_ref[...], 0.0), axis=1, keepdims=True)
        z = jnp.dot(h, w4_ref[...], preferred_element_type=jnp.float32) + b4_ref[...] + x1
        out_ref[...] = jax.nn.sigmoid(z)


def _mlp(E, w16, lane, W1, b1, W2, b2, W3, b3, W4, b4):
    return pl.pallas_call(
        _mlp_body,
        grid=(2, NT),
        in_specs=[
            pl.BlockSpec((TB, FEATP), lambda p, t: (t, 0)),
            pl.BlockSpec((TB, EDP), lambda p, t: (t, 0)),
            pl.BlockSpec((TB, 1), lambda p, t: (t, 0)),
            pl.BlockSpec((FEATP, 1024), lambda p, t: (0, 0)),
            pl.BlockSpec((1, 1024), lambda p, t: (0, 0)),
            pl.BlockSpec((1024, 512), lambda p, t: (0, 0)),
            pl.BlockSpec((1, 512), lambda p, t: (0, 0)),
            pl.BlockSpec((512, 256), lambda p, t: (0, 0)),
            pl.BlockSpec((1, 256), lambda p, t: (0, 0)),
            pl.BlockSpec((256, 1), lambda p, t: (0, 0)),
            pl.BlockSpec((1, 1), lambda p, t: (0, 0)),
        ],
        out_specs=pl.BlockSpec((TB, 1), lambda p, t: (t, 0)),
        out_shape=jax.ShapeDtypeStruct((B, 1), jnp.float32),
        scratch_shapes=[pltpu.VMEM((2, FEATP), jnp.float32)],
    )(E, w16, lane, W1, b1, W2, b2, W3, b3, W4, b4)


def kernel(x, lr_table, sparse_table, W1, b1, W2, b2, W3, b3, W4, b4):
    xi = x.astype(jnp.int32)
    idx_d = xi.reshape(NW, KTOT, G)
    x0 = xi[:, 0]
    idx_w = (x0 // EDP).reshape(NW, KW, G)
    lane = (x0 % EDP).reshape(B, 1)
    st16 = _transpose_pad(sparse_table.T)
    lr16 = lr_table[:, 0].reshape(LRR, EDP)
    e_flat, w16 = _sc_gather(idx_d, idx_w, st16, lr16)
    E = e_flat.reshape(B, FEATP)
    W1p = jnp.zeros((FEATP, 1024), W1.dtype).at[
        (jnp.arange(FEAT) // ED) * EDP + (jnp.arange(FEAT) % ED)
    ].set(W1)
    return _mlp(
        E, w16, lane,
        W1p.astype(jnp.bfloat16), b1.reshape(1, 1024),
        W2.astype(jnp.bfloat16), b2.reshape(1, 512),
        W3.astype(jnp.bfloat16), b3.reshape(1, 256),
        W4.astype(jnp.bfloat16), b4.reshape(1, 1),
    )


# speedup vs baseline: 1.8258x; 1.6084x over previous
import functools

import jax
import jax.numpy as jnp
from jax import lax
from jax.experimental import pallas as pl
from jax.experimental.pallas import tpu as pltpu
from jax.experimental.pallas import tpu_sc as plsc

B = 16384
NF = 26
ED = 10
EDP = 16
FEAT = NF * ED
FEATP = NF * EDP
NC, NS = 2, 16
NW = NC * NS
DEEP = B * NF
PER_W = DEEP // NW
G = 128
KTOT = PER_W // G
NCH = 8
KCH = KTOT // NCH
CH = PER_W // NCH
WPER = B // NW
KW = WPER // G
LRR = 1000000 // EDP


def _sc_gather(idx_d, idx_w, stable16, lr16):
    mesh = plsc.VectorSubcoreMesh(
        core_axis_name="c", subcore_axis_name="s", num_cores=NC, num_subcores=NS
    )

    @functools.partial(
        pl.kernel,
        out_type=(
            jax.ShapeDtypeStruct((DEEP, EDP), jnp.float32),
            jax.ShapeDtypeStruct((B, EDP), jnp.float32),
        ),
        mesh=mesh,
        compiler_params=pltpu.CompilerParams(use_tc_tiling_on_sc=False),
        scratch_types=(
            pltpu.VMEM((KTOT, G), jnp.int32),
            pltpu.VMEM((CH, EDP), jnp.float32),
            pltpu.VMEM((KW, G), jnp.int32),
            pltpu.VMEM((WPER, EDP), jnp.float32),
            pltpu.SemaphoreType.DMA,
        ),
    )
    def run(idx_d_hbm, idx_w_hbm, st_hbm, lr_hbm, out_e, out_w, idx_v, ebuf, widx_v, wbuf, sem):
        wid = lax.axis_index("s") * NC + lax.axis_index("c")
        pltpu.sync_copy(idx_d_hbm.at[wid], idx_v)
        pltpu.sync_copy(idx_w_hbm.at[wid], widx_v)

        wcps = [
            pltpu.make_async_copy(lr_hbm.at[widx_v.at[j]], wbuf.at[pl.ds(j * G, G)], sem)
            for j in range(KW)
        ]
        for cp in wcps:
            cp.start()
        for cp in wcps:
            cp.wait()
        pltpu.sync_copy(wbuf, out_w.at[pl.ds(wid * WPER, WPER)])

        base = wid * PER_W

        def chunk(c, carry):
            cps = [
                pltpu.make_async_copy(
                    st_hbm.at[idx_v.at[c * KCH + j]], ebuf.at[pl.ds(j * G, G)], sem
                )
                for j in range(KCH)
            ]
            for cp in cps:
                cp.start()
            for cp in cps:
                cp.wait()
            pltpu.sync_copy(ebuf, out_e.at[pl.ds(base + c * CH, CH)])
            return carry

        lax.fori_loop(0, NCH, chunk, 0)

    return run(idx_d, idx_w, stable16, lr16)


TBV = 4096


def _tp_body(in_ref, out_ref):
    xt = jnp.transpose(in_ref[...], (1, 0))
    zcol = jnp.zeros((TBV // 8, EDP - ED), jnp.float32)
    xt3 = xt.reshape(TBV // 8, 8, ED)
    for k in range(8):
        out_ref[:, k * EDP:k * EDP + ED] = xt3[:, k, :]
        out_ref[:, k * EDP + ED:(k + 1) * EDP] = zcol


def _transpose_pad(st_t):
    nv = st_t.shape[1]
    grid = (nv + TBV - 1) // TBV
    return pl.pallas_call(
        _tp_body,
        grid=(grid,),
        in_specs=[pl.BlockSpec((ED, TBV), lambda i: (0, i))],
        out_specs=pl.BlockSpec((TBV // 8, 128), lambda i: (i, 0)),
        out_shape=jax.ShapeDtypeStruct((nv // 8, 128), jnp.float32),
    )(st_t)


TB = 1024
NT = B // TB


def _mlp_body(e_ref, w16_ref, lane_ref, w1_ref, b1_ref, w2_ref, b2_ref, w3_ref,
              b3_ref, w4_ref, b4_ref, out_ref, stat_ref):
    p = pl.program_id(0)
    t = pl.program_id(1)

    @pl.when(p == 0)
    def _stats():
        e = e_ref[...]
        s = jnp.sum(e, axis=0, keepdims=True)
        s2 = jnp.sum(e * e, axis=0, keepdims=True)

        @pl.when(t == 0)
        def _init():
            stat_ref[0:1, :] = s
            stat_ref[1:2, :] = s2

        @pl.when(t > 0)
        def _acc():
            stat_ref[0:1, :] += s
            stat_ref[1:2, :] += s2

        @pl.when(t == NT - 1)
        def _fin():
            mean = stat_ref[0:1, :] * (1.0 / B)
            var = stat_ref[1:2, :] * (1.0 / B) - mean * mean
            stat_ref[0:1, :] = mean
            stat_ref[1:2, :] = lax.rsqrt(var + 1e-5)

    @pl.when(p == 1)
    def _mlp():
        mean = stat_ref[0:1, :]
        istd = stat_ref[1:2, :]
        h = ((e_ref[...] - mean) * istd).astype(jnp.bfloat16)
        h = jnp.maximum(jnp.dot(h, w1_ref[...], preferred_element_type=jnp.float32) + b1_ref[...], 0.0).astype(jnp.bfloat16)
        h = jnp.maximum(jnp.dot(h, w2_ref[...], preferred_element_type=jnp.float32) + b2_ref[...], 0.0).astype(jnp.bfloat16)
        h = jnp.maximum(jnp.dot(h, w3_ref[...], preferred_element_type=jnp.float32) + b3_ref[...], 0.0).astype(jnp.bfloat16)
        lane_hit = jax.lax.broadcasted_iota(jnp.int32, (TB, EDP), 1) == lane_ref[...]
        x1 = jnp.sum(jnp.where(lane_hit, w16_ref[...], 0.0), axis=1, keepdims=True)
        z = jnp.dot(h, w4_ref[...], preferred_element_type=jnp.float32) + b4_ref[...] + x1
        out_ref[...] = jax.nn.sigmoid(z)


def _mlp(E, w16, lane, W1, b1, W2, b2, W3, b3, W4, b4):
    return pl.pallas_call(
        _mlp_body,
        grid=(2, NT),
        in_specs=[
            pl.BlockSpec((TB, FEATP), lambda p, t: (t, 0)),
            pl.BlockSpec((TB, EDP), lambda p, t: (t, 0)),
            pl.BlockSpec((TB, 1), lambda p, t: (t, 0)),
            pl.BlockSpec((FEATP, 1024), lambda p, t: (0, 0)),
            pl.BlockSpec((1, 1024), lambda p, t: (0, 0)),
            pl.BlockSpec((1024, 512), lambda p, t: (0, 0)),
            pl.BlockSpec((1, 512), lambda p, t: (0, 0)),
            pl.BlockSpec((512, 256), lambda p, t: (0, 0)),
            pl.BlockSpec((1, 256), lambda p, t: (0, 0)),
            pl.BlockSpec((256, 1), lambda p, t: (0, 0)),
            pl.BlockSpec((1, 1), lambda p, t: (0, 0)),
        ],
        out_specs=pl.BlockSpec((TB, 1), lambda p, t: (t, 0)),
        out_shape=jax.ShapeDtypeStruct((B, 1), jnp.float32),
        scratch_shapes=[pltpu.VMEM((2, FEATP), jnp.float32)],
    )(E, w16, lane, W1, b1, W2, b2, W3, b3, W4, b4)


def kernel(x, lr_table, sparse_table, W1, b1, W2, b2, W3, b3, W4, b4):
    xi = x.astype(jnp.int32)
    idx_d = xi.reshape(NW, KTOT, G)
    x0 = xi[:, 0]
    idx_w = (x0 // EDP).reshape(NW, KW, G)
    lane = (x0 % EDP).reshape(B, 1)
    st16 = _transpose_pad(sparse_table.T).reshape(1000000, EDP)
    lr16 = lr_table[:, 0].reshape(LRR, EDP)
    e_flat, w16 = _sc_gather(idx_d, idx_w, st16, lr16)
    E = e_flat.reshape(B, FEATP)
    W1p = jnp.zeros((FEATP, 1024), W1.dtype).at[
        (jnp.arange(FEAT) // ED) * EDP + (jnp.arange(FEAT) % ED)
    ].set(W1)
    return _mlp(
        E, w16, lane,
        W1p.astype(jnp.bfloat16), b1.reshape(1, 1024),
        W2.astype(jnp.bfloat16), b2.reshape(1, 512),
        W3.astype(jnp.bfloat16), b3.reshape(1, 256),
        W4.astype(jnp.bfloat16), b4.reshape(1, 1),
    )


# speedup vs baseline: 1.8961x; 1.0385x over previous
import functools

import jax
import jax.numpy as jnp
from jax import lax
from jax.experimental import pallas as pl
from jax.experimental.pallas import tpu as pltpu
from jax.experimental.pallas import tpu_sc as plsc

B = 16384
NF = 26
ED = 10
EDP = 16
FEAT = NF * ED
FEATP = NF * EDP
NC, NS = 2, 16
NW = NC * NS
DEEP = B * NF
PER_W = DEEP // NW
G = 128
KTOT = PER_W // G
NCH = 8
KCH = KTOT // NCH
CH = PER_W // NCH
WPER = B // NW
KW = WPER // G
LRR = 1000000 // EDP


def _sc_gather(idx_d, idx_w, stable16, lr16):
    mesh = plsc.VectorSubcoreMesh(
        core_axis_name="c", subcore_axis_name="s", num_cores=NC, num_subcores=NS
    )

    @functools.partial(
        pl.kernel,
        out_type=(
            jax.ShapeDtypeStruct((DEEP, EDP), jnp.float32),
            jax.ShapeDtypeStruct((B, EDP), jnp.float32),
        ),
        mesh=mesh,
        compiler_params=pltpu.CompilerParams(use_tc_tiling_on_sc=False),
        scratch_types=(
            pltpu.VMEM((KTOT, G), jnp.int32),
            pltpu.VMEM((CH, EDP), jnp.float32),
            pltpu.VMEM((KW, G), jnp.int32),
            pltpu.VMEM((WPER, EDP), jnp.float32),
            pltpu.SemaphoreType.DMA,
        ),
    )
    def run(idx_d_hbm, idx_w_hbm, st_hbm, lr_hbm, out_e, out_w, idx_v, ebuf, widx_v, wbuf, sem):
        wid = lax.axis_index("s") * NC + lax.axis_index("c")
        pltpu.sync_copy(idx_d_hbm.at[wid], idx_v)
        pltpu.sync_copy(idx_w_hbm.at[wid], widx_v)

        wcps = [
            pltpu.make_async_copy(lr_hbm.at[widx_v.at[j]], wbuf.at[pl.ds(j * G, G)], sem)
            for j in range(KW)
        ]
        for cp in wcps:
            cp.start()
        for cp in wcps:
            cp.wait()
        pltpu.sync_copy(wbuf, out_w.at[pl.ds(wid * WPER, WPER)])

        base = wid * PER_W

        def chunk(c, carry):
            cps = [
                pltpu.make_async_copy(
                    st_hbm.at[idx_v.at[c * KCH + j]], ebuf.at[pl.ds(j * G, G)], sem
                )
                for j in range(KCH)
            ]
            for cp in cps:
                cp.start()
            for cp in cps:
                cp.wait()
            pltpu.sync_copy(ebuf, out_e.at[pl.ds(base + c * CH, CH)])
            return carry

        lax.fori_loop(0, NCH, chunk, 0)

    return run(idx_d, idx_w, stable16, lr16)


TBV = 16384


def _tp_body(in_ref, out_ref):
    xt = jnp.transpose(in_ref[...], (1, 0))
    zcol = jnp.zeros((TBV // 8, EDP - ED), jnp.float32)
    xt3 = xt.reshape(TBV // 8, 8, ED)
    for k in range(8):
        out_ref[:, k * EDP:k * EDP + ED] = xt3[:, k, :]
        out_ref[:, k * EDP + ED:(k + 1) * EDP] = zcol


def _transpose_pad(st_t):
    nv = st_t.shape[1]
    grid = (nv + TBV - 1) // TBV
    return pl.pallas_call(
        _tp_body,
        grid=(grid,),
        in_specs=[pl.BlockSpec((ED, TBV), lambda i: (0, i))],
        out_specs=pl.BlockSpec((TBV // 8, 128), lambda i: (i, 0)),
        out_shape=jax.ShapeDtypeStruct((nv // 8, 128), jnp.float32),
    )(st_t)


TB = 1024
NT = B // TB


def _mlp_body(e_ref, w16_ref, lane_ref, w1_ref, b1_ref, w2_ref, b2_ref, w3_ref,
              b3_ref, w4_ref, b4_ref, out_ref, stat_ref):
    p = pl.program_id(0)
    t = pl.program_id(1)

    @pl.when(p == 0)
    def _stats():
        e = e_ref[...]
        s = jnp.sum(e, axis=0, keepdims=True)
        s2 = jnp.sum(e * e, axis=0, keepdims=True)

        @pl.when(t == 0)
        def _init():
            stat_ref[0:1, :] = s
            stat_ref[1:2, :] = s2

        @pl.when(t > 0)
        def _acc():
            stat_ref[0:1, :] += s
            stat_ref[1:2, :] += s2

        @pl.when(t == NT - 1)
        def _fin():
            mean = stat_ref[0:1, :] * (1.0 / B)
            var = stat_ref[1:2, :] * (1.0 / B) - mean * mean
            stat_ref[0:1, :] = mean
            stat_ref[1:2, :] = lax.rsqrt(var + 1e-5)

    @pl.when(p == 1)
    def _mlp():
        mean = stat_ref[0:1, :]
        istd = stat_ref[1:2, :]
        h = ((e_ref[...] - mean) * istd).astype(jnp.bfloat16)
        h = jnp.maximum(jnp.dot(h, w1_ref[...], preferred_element_type=jnp.float32) + b1_ref[...], 0.0).astype(jnp.bfloat16)
        h = jnp.maximum(jnp.dot(h, w2_ref[...], preferred_element_type=jnp.float32) + b2_ref[...], 0.0).astype(jnp.bfloat16)
        h = jnp.maximum(jnp.dot(h, w3_ref[...], preferred_element_type=jnp.float32) + b3_ref[...], 0.0).astype(jnp.bfloat16)
        lane_hit = jax.lax.broadcasted_iota(jnp.int32, (TB, EDP), 1) == lane_ref[...]
        x1 = jnp.sum(jnp.where(lane_hit, w16_ref[...], 0.0), axis=1, keepdims=True)
        z = jnp.dot(h, w4_ref[...], preferred_element_type=jnp.float32) + b4_ref[...] + x1
        out_ref[...] = jax.nn.sigmoid(z)


def _mlp(E, w16, lane, W1, b1, W2, b2, W3, b3, W4, b4):
    return pl.pallas_call(
        _mlp_body,
        grid=(2, NT),
        in_specs=[
            pl.BlockSpec((TB, FEATP), lambda p, t: (t, 0)),
            pl.BlockSpec((TB, EDP), lambda p, t: (t, 0)),
            pl.BlockSpec((TB, 1), lambda p, t: (t, 0)),
            pl.BlockSpec((FEATP, 1024), lambda p, t: (0, 0)),
            pl.BlockSpec((1, 1024), lambda p, t: (0, 0)),
            pl.BlockSpec((1024, 512), lambda p, t: (0, 0)),
            pl.BlockSpec((1, 512), lambda p, t: (0, 0)),
            pl.BlockSpec((512, 256), lambda p, t: (0, 0)),
            pl.BlockSpec((1, 256), lambda p, t: (0, 0)),
            pl.BlockSpec((256, 1), lambda p, t: (0, 0)),
            pl.BlockSpec((1, 1), lambda p, t: (0, 0)),
        ],
        out_specs=pl.BlockSpec((TB, 1), lambda p, t: (t, 0)),
        out_shape=jax.ShapeDtypeStruct((B, 1), jnp.float32),
        scratch_shapes=[pltpu.VMEM((2, FEATP), jnp.float32)],
    )(E, w16, lane, W1, b1, W2, b2, W3, b3, W4, b4)


def kernel(x, lr_table, sparse_table, W1, b1, W2, b2, W3, b3, W4, b4):
    xi = x.astype(jnp.int32)
    idx_d = xi.reshape(NW, KTOT, G)
    x0 = xi[:, 0]
    idx_w = (x0 // EDP).reshape(NW, KW, G)
    lane = (x0 % EDP).reshape(B, 1)
    st16 = _transpose_pad(sparse_table.T).reshape(1000000, EDP)
    lr16 = lr_table[:, 0].reshape(LRR, EDP)
    e_flat, w16 = _sc_gather(idx_d, idx_w, st16, lr16)
    E = e_flat.reshape(B, FEATP)
    W1p = jnp.zeros((FEATP, 1024), W1.dtype).at[
        (jnp.arange(FEAT) // ED) * EDP + (jnp.arange(FEAT) % ED)
    ].set(W1)
    return _mlp(
        E, w16, lane,
        W1p.astype(jnp.bfloat16), b1.reshape(1, 1024),
        W2.astype(jnp.bfloat16), b2.reshape(1, 512),
        W3.astype(jnp.bfloat16), b3.reshape(1, 256),
        W4.astype(jnp.bfloat16), b4.reshape(1, 1),
    )


# speedup vs baseline: 1.9072x; 1.0059x over previous
import functools

import jax
import jax.numpy as jnp
from jax import lax
from jax.experimental import pallas as pl
from jax.experimental.pallas import tpu as pltpu
from jax.experimental.pallas import tpu_sc as plsc

B = 16384
NF = 26
ED = 10
EDP = 16
FEAT = NF * ED
FEATP = NF * EDP
NC, NS = 2, 16
NW = NC * NS
DEEP = B * NF
PER_W = DEEP // NW
G = 128
KTOT = PER_W // G
NCH = 8
KCH = KTOT // NCH
CH = PER_W // NCH
WPER = B // NW
KW = WPER // G
LRR = 1000000 // EDP


def _sc_gather(idx_d, idx_w, stable16, lr16):
    mesh = plsc.VectorSubcoreMesh(
        core_axis_name="c", subcore_axis_name="s", num_cores=NC, num_subcores=NS
    )

    @functools.partial(
        pl.kernel,
        out_type=(
            jax.ShapeDtypeStruct((DEEP, EDP), jnp.float32),
            jax.ShapeDtypeStruct((B, EDP), jnp.float32),
        ),
        mesh=mesh,
        compiler_params=pltpu.CompilerParams(use_tc_tiling_on_sc=False),
        scratch_types=(
            pltpu.VMEM((KTOT, G), jnp.int32),
            pltpu.VMEM((CH, EDP), jnp.float32),
            pltpu.VMEM((KW, G), jnp.int32),
            pltpu.VMEM((WPER, EDP), jnp.float32),
            pltpu.SemaphoreType.DMA,
        ),
    )
    def run(idx_d_hbm, idx_w_hbm, st_hbm, lr_hbm, out_e, out_w, idx_v, ebuf, widx_v, wbuf, sem):
        wid = lax.axis_index("s") * NC + lax.axis_index("c")
        pltpu.sync_copy(idx_d_hbm.at[wid], idx_v)
        pltpu.sync_copy(idx_w_hbm.at[wid], widx_v)

        wcps = [
            pltpu.make_async_copy(lr_hbm.at[widx_v.at[j]], wbuf.at[pl.ds(j * G, G)], sem)
            for j in range(KW)
        ]
        for cp in wcps:
            cp.start()
        for cp in wcps:
            cp.wait()
        pltpu.sync_copy(wbuf, out_w.at[pl.ds(wid * WPER, WPER)])

        base = wid * PER_W

        def chunk(c, carry):
            cps = [
                pltpu.make_async_copy(
                    st_hbm.at[idx_v.at[c * KCH + j]], ebuf.at[pl.ds(j * G, G)], sem
                )
                for j in range(KCH)
            ]
            for cp in cps:
                cp.start()
            for cp in cps:
                cp.wait()
            pltpu.sync_copy(ebuf, out_e.at[pl.ds(base + c * CH, CH)])
            return carry

        lax.fori_loop(0, NCH, chunk, 0)

    return run(idx_d, idx_w, stable16, lr16)


TBV = 32768


def _tp_body(in_ref, out_ref):
    xt = jnp.transpose(in_ref[...], (1, 0))
    zcol = jnp.zeros((TBV // 8, EDP - ED), jnp.float32)
    xt3 = xt.reshape(TBV // 8, 8, ED)
    for k in range(8):
        out_ref[:, k * EDP:k * EDP + ED] = xt3[:, k, :]
        out_ref[:, k * EDP + ED:(k + 1) * EDP] = zcol


def _transpose_pad(st_t):
    nv = st_t.shape[1]
    grid = (nv + TBV - 1) // TBV
    return pl.pallas_call(
        _tp_body,
        grid=(grid,),
        in_specs=[pl.BlockSpec((ED, TBV), lambda i: (0, i))],
        out_specs=pl.BlockSpec((TBV // 8, 128), lambda i: (i, 0)),
        out_shape=jax.ShapeDtypeStruct((nv // 8, 128), jnp.float32),
    )(st_t)


TB = 1024
NT = B // TB


def _mlp_body(e_ref, w16_ref, lane_ref, w1_ref, b1_ref, w2_ref, b2_ref, w3_ref,
              b3_ref, w4_ref, b4_ref, out_ref, stat_ref):
    p = pl.program_id(0)
    t = pl.program_id(1)

    @pl.when(p == 0)
    def _stats():
        e = e_ref[...]
        s = jnp.sum(e, axis=0, keepdims=True)
        s2 = jnp.sum(e * e, axis=0, keepdims=True)

        @pl.when(t == 0)
        def _init():
            stat_ref[0:1, :] = s
            stat_ref[1:2, :] = s2

        @pl.when(t > 0)
        def _acc():
            stat_ref[0:1, :] += s
            stat_ref[1:2, :] += s2

        @pl.when(t == NT - 1)
        def _fin():
            mean = stat_ref[0:1, :] * (1.0 / B)
            var = stat_ref[1:2, :] * (1.0 / B) - mean * mean
            stat_ref[0:1, :] = mean
            stat_ref[1:2, :] = lax.rsqrt(var + 1e-5)

    @pl.when(p == 1)
    def _mlp():
        mean = stat_ref[0:1, :]
        istd = stat_ref[1:2, :]
        h = ((e_ref[...] - mean) * istd).astype(jnp.bfloat16)
        h = jnp.maximum(jnp.dot(h, w1_ref[...], preferred_element_type=jnp.float32) + b1_ref[...], 0.0).astype(jnp.bfloat16)
        h = jnp.maximum(jnp.dot(h, w2_ref[...], preferred_element_type=jnp.float32) + b2_ref[...], 0.0).astype(jnp.bfloat16)
        h = jnp.maximum(jnp.dot(h, w3_ref[...], preferred_element_type=jnp.float32) + b3_ref[...], 0.0).astype(jnp.bfloat16)
        lane_hit = jax.lax.broadcasted_iota(jnp.int32, (TB, EDP), 1) == lane_ref[...]
        x1 = jnp.sum(jnp.where(lane_hit, w16_ref[...], 0.0), axis=1, keepdims=True)
        z = jnp.dot(h, w4_ref[...], preferred_element_type=jnp.float32) + b4_ref[...] + x1
        out_ref[...] = jax.nn.sigmoid(z)


def _mlp(E, w16, lane, W1, b1, W2, b2, W3, b3, W4, b4):
    return pl.pallas_call(
        _mlp_body,
        grid=(2, NT),
        in_specs=[
            pl.BlockSpec((TB, FEATP), lambda p, t: (t, 0)),
            pl.BlockSpec((TB, EDP), lambda p, t: (t, 0)),
            pl.BlockSpec((TB, 1), lambda p, t: (t, 0)),
            pl.BlockSpec((FEATP, 1024), lambda p, t: (0, 0)),
            pl.BlockSpec((1, 1024), lambda p, t: (0, 0)),
            pl.BlockSpec((1024, 512), lambda p, t: (0, 0)),
            pl.BlockSpec((1, 512), lambda p, t: (0, 0)),
            pl.BlockSpec((512, 256), lambda p, t: (0, 0)),
            pl.BlockSpec((1, 256), lambda p, t: (0, 0)),
            pl.BlockSpec((256, 1), lambda p, t: (0, 0)),
            pl.BlockSpec((1, 1), lambda p, t: (0, 0)),
        ],
        out_specs=pl.BlockSpec((TB, 1), lambda p, t: (t, 0)),
        out_shape=jax.ShapeDtypeStruct((B, 1), jnp.float32),
        scratch_shapes=[pltpu.VMEM((2, FEATP), jnp.float32)],
    )(E, w16, lane, W1, b1, W2, b2, W3, b3, W4, b4)


def kernel(x, lr_table, sparse_table, W1, b1, W2, b2, W3, b3, W4, b4):
    xi = x.astype(jnp.int32)
    idx_d = xi.reshape(NW, KTOT, G)
    x0 = xi[:, 0]
    idx_w = (x0 // EDP).reshape(NW, KW, G)
    lane = (x0 % EDP).reshape(B, 1)
    st16 = _transpose_pad(sparse_table.T).reshape(1000000, EDP)
    lr16 = lr_table[:, 0].reshape(LRR, EDP)
    e_flat, w16 = _sc_gather(idx_d, idx_w, st16, lr16)
    E = e_flat.reshape(B, FEATP)
    W1p = jnp.zeros((FEATP, 1024), W1.dtype).at[
        (jnp.arange(FEAT) // ED) * EDP + (jnp.arange(FEAT) % ED)
    ].set(W1)
    return _mlp(
        E, w16, lane,
        W1p.astype(jnp.bfloat16), b1.reshape(1, 1024),
        W2.astype(jnp.bfloat16), b2.reshape(1, 512),
        W3.astype(jnp.bfloat16), b3.reshape(1, 256),
        W4.astype(jnp.bfloat16), b4.reshape(1, 1),
    )
